# baseline (speedup 1.0000x reference)
import functools

import jax
import jax.numpy as jnp
from jax import lax
from jax.experimental import pallas as pl
from jax.experimental.pallas import tpu as pltpu
from jax.experimental.pallas import tpu_sc as plsc

B, N, C = 64, 1024, 192
LAST_TILE = (N + 2) // 128
LAST_OFF = (N + 1) % 128
B_BLK = 8
L = 16
NCHUNK = N // L
NW = 32
ROWS_PW = B // NW


def _logits_body(vtail_ref, tb_ref, wt_ref, bias_ref, keys_ref, kvec_ref):
    br = vtail_ref[:, :, LAST_OFF]
    logits = jax.lax.dot_general(
        br, wt_ref[...],
        dimension_numbers=(((1,), (0,)), ((), ())),
        preferred_element_type=jnp.float32) + bias_ref[...]

    x = logits + 0.0
    u = jax.lax.bitcast_convert_type(x, jnp.uint32)
    keys_ref[...] = u ^ jnp.where(
        (u >> 31) > 0, jnp.uint32(0xFFFFFFFF), jnp.uint32(0x80000000))

    ii = jax.lax.broadcasted_iota(jnp.int32, (B, B), 0)
    jj = jax.lax.broadcasted_iota(jnp.int32, (B, B), 1)
    tb_col = jnp.sum(jnp.where(ii == jj, jnp.broadcast_to(tb_ref[...], (B, B)),
                               0.0), axis=1, keepdims=True)
    K = jnp.clip((tb_col * float(N)).astype(jnp.int32), 1, N)
    kvec_ref[...] = jnp.broadcast_to(K, (B, 128))


_sc_mesh = plsc.VectorSubcoreMesh(core_axis_name="c", subcore_axis_name="s")


@functools.partial(
    pl.kernel,
    mesh=_sc_mesh,
    compiler_params=pltpu.CompilerParams(needs_layout_passes=False),
    out_type=jax.ShapeDtypeStruct((B, N), jnp.float32),
    scratch_types=[
        pltpu.VMEM((ROWS_PW, N), jnp.uint32),
        pltpu.VMEM((ROWS_PW, 128), jnp.int32),
        pltpu.VMEM((ROWS_PW, N), jnp.float32),
    ],
)
def _sc_mask(keys_hbm, kvec_hbm, mask_hbm, keys_v, kv_v, out_v):
    cid = lax.axis_index("c")
    sid = lax.axis_index("s")
    wid = sid * 2 + cid
    base = wid * ROWS_PW
    pltpu.sync_copy(kvec_hbm.at[pl.ds(base, ROWS_PW)], kv_v)
    pltpu.sync_copy(keys_hbm.at[pl.ds(base, ROWS_PW)], keys_v)

    for rr in range(ROWS_PW):
        K = kv_v[rr, pl.ds(0, L)]

        def bit_step(i, t):
            sh = jnp.broadcast_to(31 - i, (L,)).astype(jnp.uint32)
            cand = t | (jnp.full((L,), 1, jnp.uint32) << sh)
            cnt = jnp.zeros((L,), jnp.int32)
            for c in range(NCHUNK):
                v = keys_v[rr, pl.ds(c * L, L)]
                cnt = cnt + plsc.all_reduce_population_count(v >= cand)
            return jnp.where(cnt >= K, cand, t)

        t = lax.fori_loop(0, 32, bit_step, jnp.zeros((L,), jnp.uint32))

        gt_cnt = jnp.zeros((L,), jnp.int32)
        for c in range(NCHUNK):
            v = keys_v[rr, pl.ds(c * L, L)]
            gt_cnt = gt_cnt + plsc.all_reduce_population_count(v > t)
        need = K - gt_cnt

        carry = jnp.zeros((L,), jnp.int32)
        one = jnp.full((L,), 1.0, jnp.float32)
        zero = jnp.zeros((L,), jnp.float32)
        for c in range(NCHUNK):
            v = keys_v[rr, pl.ds(c * L, L)]
            gt = v > t
            eq = v == t
            eqi = jnp.where(eq, 1, 0)
            excl = plsc.cumsum(eqi) - eqi
            keep = gt | (eq & ((carry + excl) < need))
            carry = carry + plsc.all_reduce_population_count(eq)
            out_v[rr, pl.ds(c * L, L)] = jnp.where(keep, one, zero)

    pltpu.sync_copy(out_v, mask_hbm.at[pl.ds(base, ROWS_PW)])


def _mul_body(vt_ref, mask_ref, out_ref):
    patches = vt_ref[:, :, pl.ds(1, N)]
    m = mask_ref[...]
    out_ref[...] = patches * m[:, None, :]


def kernel(vision_output, token_budget, W, b):
    vt = jnp.transpose(vision_output, (0, 2, 1))
    b2 = b.reshape(1, N)
    wt = W.T

    keys, kvec = pl.pallas_call(
        _logits_body,
        grid=(1,),
        in_specs=[
            pl.BlockSpec((B, C, 128), lambda i: (0, 0, LAST_TILE)),
            pl.BlockSpec((1, B), lambda i: (0, 0)),
            pl.BlockSpec((C, N), lambda i: (0, 0)),
            pl.BlockSpec((1, N), lambda i: (0, 0)),
        ],
        out_specs=[
            pl.BlockSpec((B, N), lambda i: (0, 0)),
            pl.BlockSpec((B, 128), lambda i: (0, 0)),
        ],
        out_shape=[
            jax.ShapeDtypeStruct((B, N), jnp.uint32),
            jax.ShapeDtypeStruct((B, 128), jnp.int32),
        ],
    )(vt, token_budget.reshape(1, B), wt, b2)

    keep_mask = _sc_mask(keys, kvec)

    masked_t = pl.pallas_call(
        _mul_body,
        grid=(B // B_BLK,),
        in_specs=[
            pl.BlockSpec((B_BLK, C, N + 2), lambda i: (i, 0, 0)),
            pl.BlockSpec((B_BLK, N), lambda i: (i, 0)),
        ],
        out_specs=pl.BlockSpec((B_BLK, C, N), lambda i: (i, 0, 0)),
        out_shape=jax.ShapeDtypeStruct((B, C, N), jnp.float32),
    )(vt, keep_mask)

    masked = jnp.transpose(masked_t, (0, 2, 1))
    return masked, keep_mask
